# baseline (speedup 1.0000x reference)
import functools

import jax
import jax.numpy as jnp
from jax import lax
from jax.experimental import pallas as pl
from jax.experimental.pallas import tpu as pltpu
from jax.experimental.pallas import tpu_sc as plsc

QUARTER = 192
SEQ = 81
HID = 4 * QUARTER
BS = 3
SC_ROWS = 30
TC_ROWS = SEQ - SC_ROWS
REP = 128
NC = 2
NS = 16
NW = NC * NS


def _enc_from_tables(row, col, box, pos):
    row81 = jnp.broadcast_to(row[:, None, :], (9, 9, QUARTER)).reshape(SEQ, QUARTER)
    col81 = jnp.broadcast_to(col[None, :, :], (9, 9, QUARTER)).reshape(SEQ, QUARTER)
    boxr = box.reshape(3, 3, QUARTER)
    box81 = jnp.broadcast_to(
        boxr[:, None, :, None, :], (3, 3, 3, 3, QUARTER)
    ).reshape(SEQ, QUARTER)
    return jnp.concatenate([row81, col81, box81, pos], axis=-1)


def _enc_rep_kernel(row_ref, col_ref, box_ref, pos_ref, rep_ref):
    enc = _enc_from_tables(row_ref[:], col_ref[:], box_ref[:], pos_ref[:])
    tail = enc[TC_ROWS:, :]
    rep_ref[:] = jnp.broadcast_to(tail[:, None, :], (SC_ROWS, REP, HID))


def _one_hot_rows(idx, n, table):
    j = jax.lax.broadcasted_iota(jnp.int32, (BS, n), 1)
    oh = (j == idx[:, None]).astype(jnp.float32)
    return jax.lax.dot_general(
        oh, table, (((1,), (0,)), ((), ())),
        preferred_element_type=jnp.float32)


def _bcast_kernel(batch, row_ref, col_ref, box_ref, pos_ref, out_ref):
    i = pl.program_id(0)
    p = i * BS + jax.lax.broadcasted_iota(jnp.int32, (BS,), 0)
    r, c = p // 9, p % 9
    b = (r // 3) * 3 + c // 3
    rows = jnp.concatenate([
        _one_hot_rows(r, 9, row_ref[:]),
        _one_hot_rows(c, 9, col_ref[:]),
        _one_hot_rows(b, 9, box_ref[:]),
        _one_hot_rows(p, SEQ, pos_ref[:]),
    ], axis=-1)
    out_ref[:] = jnp.broadcast_to(rows[:, None, :], (BS, batch, HID))


def _sc_bcast(batch, rep_hbm, out_hbm, slab_v, in_sem, out_sem):
    wid = lax.axis_index("s") * NC + lax.axis_index("c")

    @pl.when(wid < SC_ROWS)
    def _():
        pltpu.make_async_copy(rep_hbm.at[wid], slab_v, in_sem).start()
        pltpu.make_async_copy(rep_hbm.at[wid], slab_v, in_sem).wait()
        outs = [
            pltpu.make_async_copy(
                slab_v, out_hbm.at[wid, pl.ds(k * REP, REP)], out_sem)
            for k in range(batch // REP)
        ]
        for cp in outs:
            cp.start()
        for cp in outs:
            cp.wait()


@functools.partial(jax.jit, static_argnames=("batch",))
def _run(row_embed, col_embed, box_embed, pos_embed, batch):
    assert batch % REP == 0
    enc_rep = pl.pallas_call(
        _enc_rep_kernel,
        out_shape=jax.ShapeDtypeStruct((SC_ROWS, REP, HID), jnp.float32),
    )(row_embed, col_embed, box_embed, pos_embed)

    mesh = plsc.VectorSubcoreMesh(core_axis_name="c", subcore_axis_name="s")
    sc_out = pl.kernel(
        functools.partial(_sc_bcast, batch),
        out_type=jax.ShapeDtypeStruct((SC_ROWS, batch, HID), jnp.float32),
        mesh=mesh,
        scratch_types=[
            pltpu.VMEM((REP, HID), jnp.float32),
            pltpu.SemaphoreType.DMA,
            pltpu.SemaphoreType.DMA,
        ],
        compiler_params=pltpu.CompilerParams(use_tc_tiling_on_sc=True),
    )(enc_rep)

    tc_out = pl.pallas_call(
        functools.partial(_bcast_kernel, batch),
        grid=(TC_ROWS // BS,),
        in_specs=[
            pl.BlockSpec((9, QUARTER), lambda i: (0, 0)),
            pl.BlockSpec((9, QUARTER), lambda i: (0, 0)),
            pl.BlockSpec((9, QUARTER), lambda i: (0, 0)),
            pl.BlockSpec((SEQ, QUARTER), lambda i: (0, 0)),
        ],
        out_specs=pl.BlockSpec((BS, batch, HID), lambda i: (i, 0, 0)),
        out_shape=jax.ShapeDtypeStruct((TC_ROWS, batch, HID), jnp.float32),
        compiler_params=pltpu.CompilerParams(
            dimension_semantics=("parallel",),
        ),
    )(row_embed, col_embed, box_embed, pos_embed)

    full = jnp.concatenate([tc_out, sc_out], axis=0)
    return jnp.transpose(full, (1, 0, 2))


def kernel(x, row_embed, col_embed, box_embed, pos_embed):
    batch = x.shape[0]
    return _run(row_embed, col_embed, box_embed, pos_embed, batch)


# speedup vs baseline: 3.3166x; 3.3166x over previous
import functools

import jax
import jax.numpy as jnp
from jax.experimental import pallas as pl
from jax.experimental.pallas import tpu as pltpu

QUARTER = 192
SEQ = 81
HID = 4 * QUARTER
BS = 3


def _one_hot_rows(idx, n, table):
    j = jax.lax.broadcasted_iota(jnp.int32, (BS, n), 1)
    oh = (j == idx[:, None]).astype(jnp.float32)
    return jax.lax.dot_general(
        oh, table, (((1,), (0,)), ((), ())),
        preferred_element_type=jnp.float32)


def _bcast_kernel(batch, row_ref, col_ref, box_ref, pos_ref, out_ref):
    i = pl.program_id(0)
    p = i * BS + jax.lax.broadcasted_iota(jnp.int32, (BS,), 0)
    r, c = p // 9, p % 9
    b = (r // 3) * 3 + c // 3
    rows = jnp.concatenate([
        _one_hot_rows(r, 9, row_ref[:]),
        _one_hot_rows(c, 9, col_ref[:]),
        _one_hot_rows(b, 9, box_ref[:]),
        _one_hot_rows(p, SEQ, pos_ref[:]),
    ], axis=-1)
    out_ref[:] = jnp.broadcast_to(rows[:, None, :], (BS, batch, HID))


@functools.partial(jax.jit, static_argnames=("batch",))
def _run(row_embed, col_embed, box_embed, pos_embed, batch):
    grid = (SEQ // BS,)
    out = pl.pallas_call(
        functools.partial(_bcast_kernel, batch),
        grid=grid,
        in_specs=[
            pl.BlockSpec((9, QUARTER), lambda i: (0, 0)),
            pl.BlockSpec((9, QUARTER), lambda i: (0, 0)),
            pl.BlockSpec((9, QUARTER), lambda i: (0, 0)),
            pl.BlockSpec((SEQ, QUARTER), lambda i: (0, 0)),
        ],
        out_specs=pl.BlockSpec((BS, batch, HID), lambda i: (i, 0, 0)),
        out_shape=jax.ShapeDtypeStruct((SEQ, batch, HID), jnp.float32),
        compiler_params=pltpu.CompilerParams(
            dimension_semantics=("parallel",),
        ),
    )(row_embed, col_embed, box_embed, pos_embed)
    return jnp.transpose(out, (1, 0, 2))


def kernel(x, row_embed, col_embed, box_embed, pos_embed):
    batch = x.shape[0]
    return _run(row_embed, col_embed, box_embed, pos_embed, batch)


# speedup vs baseline: 3.3728x; 1.0169x over previous
import functools

import jax
import jax.numpy as jnp
from jax.experimental import pallas as pl
from jax.experimental.pallas import tpu as pltpu

QUARTER = 192
SEQ = 81
HID = 4 * QUARTER
BS = 3


def _one_hot_rows(idx, n, table):
    j = jax.lax.broadcasted_iota(jnp.int32, (BS, n), 1)
    oh = (j == idx[:, None]).astype(jnp.float32)
    return jax.lax.dot_general(
        oh, table, (((1,), (0,)), ((), ())),
        preferred_element_type=jnp.float32)


def _bcast_kernel(batch, row_ref, col_ref, box_ref, pos_ref, out_ref):
    i = pl.program_id(0)
    p = i * BS + jax.lax.broadcasted_iota(jnp.int32, (BS,), 0)
    r, c = p // 9, p % 9
    b = (r // 3) * 3 + c // 3
    rows = jnp.concatenate([
        _one_hot_rows(r, 9, row_ref[:]),
        _one_hot_rows(c, 9, col_ref[:]),
        _one_hot_rows(b, 9, box_ref[:]),
        _one_hot_rows(p, SEQ, pos_ref[:]),
    ], axis=-1)
    out_ref[:] = jnp.broadcast_to(rows[:, None, :], (BS, batch // 2, HID))


@functools.partial(jax.jit, static_argnames=("batch",))
def _run(row_embed, col_embed, box_embed, pos_embed, batch):
    grid = (SEQ // BS,)
    out = pl.pallas_call(
        functools.partial(_bcast_kernel, batch),
        grid=(SEQ // BS, 2),
        in_specs=[
            pl.BlockSpec((9, QUARTER), lambda i, j: (0, 0)),
            pl.BlockSpec((9, QUARTER), lambda i, j: (0, 0)),
            pl.BlockSpec((9, QUARTER), lambda i, j: (0, 0)),
            pl.BlockSpec((SEQ, QUARTER), lambda i, j: (0, 0)),
        ],
        out_specs=pl.BlockSpec((BS, batch // 2, HID), lambda i, j: (i, j, 0)),
        out_shape=jax.ShapeDtypeStruct((SEQ, batch, HID), jnp.float32),
        compiler_params=pltpu.CompilerParams(
            dimension_semantics=("parallel", "parallel"),
        ),
    )(row_embed, col_embed, box_embed, pos_embed)
    return jnp.transpose(out, (1, 0, 2))


def kernel(x, row_embed, col_embed, box_embed, pos_embed):
    batch = x.shape[0]
    return _run(row_embed, col_embed, box_embed, pos_embed, batch)
